# Initial kernel scaffold; baseline (speedup 1.0000x reference)
import jax
import jax.numpy as jnp
from jax.experimental import pallas as pl


def kernel(q, k, v):
    raise NotImplementedError("write your pallas kernel here")


# speedup vs baseline: 3.1637x; 3.1637x over previous
import functools

import jax
import jax.numpy as jnp
import numpy as np
from jax import lax
from jax.experimental import pallas as pl
from jax.experimental.pallas import tpu as pltpu
from jax.experimental.pallas import tpu_sc as plsc

B, C, N = 2, 1024, 4096
GROUP_SIZES = (128, 128, 256, 512)
GROUP_STARTS = (0, 128, 256, 512)

NC, NS = 2, 16
NW = NC * NS
ROWS_PER_LEAF_PER_W = tuple(2 * g // NW for g in GROUP_SIZES)
CHUNK = 8


def _rank_body(e_ref, idx_ref):
    e = e_ref[...]
    e_lanes = jnp.broadcast_to(e, (C, C))
    e_rows = lax.broadcast_in_dim(e.reshape(C), (C, C), (0,))
    ii = lax.broadcasted_iota(jnp.int32, (C, C), 0)
    jj = lax.broadcasted_iota(jnp.int32, (C, C), 1)
    before = (e_rows > e_lanes) | ((e_rows == e_lanes) & (ii < jj))
    rank = jnp.sum(before.astype(jnp.int32), axis=0)
    rank_lanes = jnp.broadcast_to(rank.reshape(1, C), (C, C))
    rr = lax.broadcasted_iota(jnp.int32, (C, C), 0)
    onehot = (rank_lanes == rr).astype(jnp.int32)
    sorted_idx = jnp.sum(onehot * jj, axis=1)
    idx_ref[...] = sorted_idx.reshape(C, 1)


def _sorted_idx_from_energy(e):
    return pl.pallas_call(
        _rank_body,
        out_shape=jax.ShapeDtypeStruct((C, 1), jnp.int32),
    )(e.reshape(1, C)).reshape(C)


def _gather_kernel(q_hbm, k_hbm, v_hbm, gidx_hbm,
                   q0, q1, q2, q3, k0, k1, k2, k3, v0, v1, v2, v3,
                   idx_v, buf0, buf1, sem0, sem1):
    wid = lax.axis_index("s") * NC + lax.axis_index("c")
    outs = ((q0, k0, v0), (q1, k1, v1), (q2, k2, v2), (q3, k3, v3))
    tables = (q_hbm, k_hbm, v_hbm)

    off = 0
    for g in range(4):
        rows_pw = ROWS_PER_LEAF_PER_W[g]
        base = 2 * GROUP_STARTS[g] + wid * rows_pw
        pltpu.sync_copy(gidx_hbm.at[pl.ds(base, rows_pw)],
                        idx_v.at[pl.ds(off, rows_pw)])
        off += rows_pw

    sched = []
    off = 0
    for g in range(4):
        rows_pw = ROWS_PER_LEAF_PER_W[g]
        for c in range(rows_pw // CHUNK):
            sched.append((g, c, off + c * CHUNK))
        off += rows_pw

    bufs = (buf0, buf1)
    sems = (sem0, sem1)
    for t in range(3):
        table = tables[t]
        handles = [None, None]
        for s, (g, c, ioff) in enumerate(sched):
            p = s % 2
            handles[p] = pltpu.async_copy(
                table.at[idx_v.at[pl.ds(ioff, CHUNK)]], bufs[p], sems[p])
            if s > 0:
                pg, pc, _ = sched[s - 1]
                prev = (s - 1) % 2
                handles[prev].wait()
                row0 = wid * ROWS_PER_LEAF_PER_W[pg] + pc * CHUNK
                pltpu.sync_copy(bufs[prev], outs[pg][t].at[pl.ds(row0, CHUNK)])
        g, c, _ = sched[-1]
        p = (len(sched) - 1) % 2
        handles[p].wait()
        row0 = wid * ROWS_PER_LEAF_PER_W[g] + c * CHUNK
        pltpu.sync_copy(bufs[p], outs[g][t].at[pl.ds(row0, CHUNK)])


def kernel(q, k, v):
    energy = jnp.mean(jnp.mean(jnp.abs(q), axis=-1), axis=0)
    sorted_idx = _sorted_idx_from_energy(energy)

    parts = []
    idx_groups = []
    for g, s0 in zip(GROUP_SIZES, GROUP_STARTS):
        sl = lax.slice(sorted_idx, (s0,), (s0 + g,))
        idx_groups.append(sl)
        parts.append(sl)
        parts.append(sl + C)
    gidx = jnp.concatenate(parts)

    out_type = [jax.ShapeDtypeStruct((2 * g, N), jnp.float32)
                for g in GROUP_SIZES] * 3

    @functools.partial(
        pl.kernel,
        mesh=plsc.VectorSubcoreMesh(core_axis_name="c", subcore_axis_name="s"),
        out_type=out_type,
        scratch_types=[
            pltpu.VMEM((2 * C // NW,), jnp.int32),
            pltpu.VMEM((CHUNK, N), jnp.float32),
            pltpu.VMEM((CHUNK, N), jnp.float32),
            pltpu.SemaphoreType.DMA,
            pltpu.SemaphoreType.DMA,
        ],
    )
    def gather_call(q_hbm, k_hbm, v_hbm, gidx_hbm, *refs):
        qo = refs[0:4]
        ko = refs[4:8]
        vo = refs[8:12]
        idx_v, buf0, buf1, sem0, sem1 = refs[12:17]
        _gather_kernel(q_hbm, k_hbm, v_hbm, gidx_hbm,
                       qo[0], qo[1], qo[2], qo[3],
                       ko[0], ko[1], ko[2], ko[3],
                       vo[0], vo[1], vo[2], vo[3],
                       idx_v, buf0, buf1, sem0, sem1)

    flat = gather_call(q.reshape(B * C, N), k.reshape(B * C, N),
                       v.reshape(B * C, N), gidx)
    q_groups = [flat[i].reshape(B, g, N) for i, g in enumerate(GROUP_SIZES)]
    k_groups = [flat[4 + i].reshape(B, g, N) for i, g in enumerate(GROUP_SIZES)]
    v_groups = [flat[8 + i].reshape(B, g, N) for i, g in enumerate(GROUP_SIZES)]
    return tuple(q_groups) + tuple(k_groups) + tuple(v_groups) + tuple(idx_groups)
